# baseline (speedup 1.0000x reference)
import functools

import jax
import jax.numpy as jnp
from jax import lax
from jax.experimental import pallas as pl
from jax.experimental.pallas import tpu as pltpu
from jax.experimental.pallas import tpu_sc as plsc

POSITIONS = 200
BATCH = 16384
NLANES = 16
NW = 32


def _sc_body(in_hbm, out_hbm, idx_v, row_v):
    cid = lax.axis_index("c")
    sid = lax.axis_index("s")
    wid = cid * 16 + sid
    row_start = wid * 6 + jnp.minimum(wid, 8)
    nrows = jnp.where(wid < 8, 7, 6)

    pltpu.sync_copy(in_hbm, idx_v)

    def do_row(j_local, _):
        j = row_start + j_local

        @pl.when(j_local < nrows)
        def _():
            def chunk(c, _):
                v = idx_v[pl.ds(c * NLANES, NLANES)]
                row_v[pl.ds(c * NLANES, NLANES)] = jnp.where(
                    v == j, 1.0, 0.0
                ).astype(jnp.float32)
                return 0

            lax.fori_loop(0, BATCH // NLANES, chunk, 0, unroll=8)
            pltpu.sync_copy(row_v, out_hbm.at[j])

        return 0

    lax.fori_loop(0, 7, do_row, 0)


def kernel(inputs):
    mesh = plsc.VectorSubcoreMesh(core_axis_name="c", subcore_axis_name="s")
    out_t = functools.partial(
        pl.kernel,
        mesh=mesh,
        out_type=jax.ShapeDtypeStruct((POSITIONS, BATCH), jnp.float32),
        scratch_types=[
            pltpu.VMEM((BATCH,), jnp.int32),
            pltpu.VMEM((BATCH,), jnp.float32),
        ],
    )(_sc_body)(inputs)
    return out_t.T


# speedup vs baseline: 10.2599x; 10.2599x over previous
import jax
import jax.numpy as jnp
from jax.experimental import pallas as pl
from jax.experimental.pallas import tpu as pltpu

POSITIONS = 200
CHUNK = 4096
NCHUNK = 4


def _onehot_t_block(in_ref, out_ref):
    vals = in_ref[0, 0, :]
    rows = jax.lax.broadcasted_iota(jnp.int32, (POSITIONS, CHUNK), 0)
    out_ref[:, :] = (vals[None, :] == rows).astype(jnp.float32)


def kernel(inputs):
    n = inputs.shape[0]
    inputs3 = inputs.reshape(NCHUNK, 1, CHUNK)
    out_t = pl.pallas_call(
        _onehot_t_block,
        grid=(NCHUNK,),
        in_specs=[pl.BlockSpec((1, 1, CHUNK), lambda i: (i, 0, 0))],
        out_specs=pl.BlockSpec((POSITIONS, CHUNK), lambda i: (0, i)),
        out_shape=jax.ShapeDtypeStruct((POSITIONS, n), jnp.float32),
        compiler_params=pltpu.CompilerParams(
            dimension_semantics=("parallel",),
        ),
    )(inputs3)
    return out_t.T
